# baseline (speedup 1.0000x reference)
import functools

import jax
import jax.numpy as jnp
from jax import lax
from jax.experimental import pallas as pl
from jax.experimental.pallas import tpu as pltpu

Q = 4096
D = 128
K = 100000
BK = 512
K_PAD = ((K + BK - 1) // BK) * BK
NK = K_PAD // BK
SUPER = 1
PAD_VAL = 3.0e38
INT_BIG = 2**30


def _rne_bf16(v):
    r = lax.bitcast_convert_type(v, jnp.int32)
    lsb = jnp.bitwise_and(lax.shift_right_logical(r, 16), 1)
    r = r + 0x7FFF + lsb
    r = jnp.bitwise_and(r, jnp.int32(-65536))
    return lax.bitcast_convert_type(r, jnp.float32)


def _argmin_body(x_ref, m_ref, xsq_ref, msq_ref, idx_out_ref,
                 runv_ref, runi_ref, accv_ref):
    k = pl.program_id(0)

    @pl.when(k == 0)
    def _init():
        runv_ref[...] = jnp.full((Q, 1), jnp.float32(PAD_VAL))
        runi_ref[...] = jnp.zeros((Q, 1), jnp.int32)
        accv_ref[...] = jnp.full((Q, 1), jnp.float32(PAD_VAL))
        idx_out_ref[...] = jnp.zeros((Q, 1), jnp.int32)

    dots = lax.dot_general(
        x_ref[...], m_ref[...], (((1,), (1,)), ((), ())),
        preferred_element_type=jnp.float32,
    )
    scores = (xsq_ref[...] - 2.0 * dots) + msq_ref[0]

    blk_min = jnp.min(scores, axis=1, keepdims=True)
    ids = lax.broadcasted_iota(jnp.int32, (Q, BK), 1) + k * BK
    blk_arg = jnp.min(
        jnp.where(scores == blk_min, ids, INT_BIG), axis=1, keepdims=True)

    rv, ri = runv_ref[...], runi_ref[...]
    better = (blk_min < rv) | ((blk_min == rv) & (blk_arg < ri))
    rv = jnp.where(better, blk_min, rv)
    ri = jnp.where(better, blk_arg, ri)
    runv_ref[...] = rv
    runi_ref[...] = ri

    @pl.when((k % SUPER == SUPER - 1) | (k == NK - 1))
    def _fold():
        av, ai = accv_ref[...], idx_out_ref[...]
        b = (rv < av) | ((rv == av) & (ri < ai))
        accv_ref[...] = _rne_bf16(jnp.where(b, rv, av))
        idx_out_ref[...] = jnp.where(b, ri, ai)
        runv_ref[...] = jnp.full((Q, 1), jnp.float32(PAD_VAL))
        runi_ref[...] = jnp.zeros((Q, 1), jnp.int32)


def _nn_indices(X, Mpad, x_sq, m_sq3):
    return pl.pallas_call(
        _argmin_body,
        grid=(NK,),
        in_specs=[
            pl.BlockSpec((Q, D), lambda k: (0, 0)),
            pl.BlockSpec((BK, D), lambda k: (k, 0)),
            pl.BlockSpec((Q, 1), lambda k: (0, 0)),
            pl.BlockSpec((1, 1, BK), lambda k: (k, 0, 0)),
        ],
        out_specs=pl.BlockSpec((Q, 1), lambda k: (0, 0)),
        out_shape=jax.ShapeDtypeStruct((Q, 1), jnp.int32),
        scratch_shapes=[
            pltpu.VMEM((Q, 1), jnp.float32),
            pltpu.VMEM((Q, 1), jnp.int32),
            pltpu.VMEM((Q, 1), jnp.float32),
        ],
        compiler_params=pltpu.CompilerParams(
            dimension_semantics=("arbitrary",),
        ),
    )(X, Mpad, x_sq, m_sq3)


def _make_sc_gather():
    from jax.experimental.pallas import tpu_sc as plsc

    info = plsc.get_sparse_core_info()
    nw = info.num_cores * info.num_subcores
    b_per_w = Q // nw
    mesh = plsc.VectorSubcoreMesh(core_axis_name="c", subcore_axis_name="s")

    @functools.partial(
        pl.kernel,
        mesh=mesh,
        out_type=jax.ShapeDtypeStruct((Q, D), jnp.float32),
        scratch_types=[
            pltpu.VMEM((b_per_w,), jnp.int32),
            pltpu.VMEM((b_per_w, D), jnp.float32),
            pltpu.SemaphoreType.DMA,
        ],
    )
    def gather(table_hbm, idx_hbm, out_hbm, idx_v, rows_v, sem):
        wid = lax.axis_index("s") * info.num_cores + lax.axis_index("c")
        base = wid * b_per_w
        pltpu.sync_copy(idx_hbm.at[pl.ds(base, b_per_w)], idx_v)
        pltpu.async_copy(table_hbm.at[idx_v], rows_v, sem).wait()
        pltpu.sync_copy(rows_v, out_hbm.at[pl.ds(base, b_per_w)])

    return gather


_sc_gather = None


def kernel(X, manifold_ptsX):
    global _sc_gather
    if _sc_gather is None:
        _sc_gather = _make_sc_gather()
    Mpad = jnp.pad(manifold_ptsX, ((0, K_PAD - K), (0, 0)))
    x_sq = jnp.sum(X * X, axis=1, keepdims=True)
    m_sq = jnp.sum(manifold_ptsX * manifold_ptsX, axis=1)
    m_sq3 = jnp.pad(m_sq, (0, K_PAD - K),
                    constant_values=PAD_VAL).reshape(NK, 1, BK)
    idx = _nn_indices(X, Mpad, x_sq, m_sq3).reshape((Q,))
    return _sc_gather(manifold_ptsX, idx)
